# Initial kernel scaffold; baseline (speedup 1.0000x reference)
import jax
import jax.numpy as jnp
from jax.experimental import pallas as pl


def kernel(in_feat, edge_index, edge_weight, W1, b1, W2, b2, W3, b3, W4, b4, W5, b5, W6, b6, Wl, bl):
    raise NotImplementedError("write your pallas kernel here")


# speedup vs baseline: 1.6426x; 1.6426x over previous
import functools

import jax
import jax.numpy as jnp
from jax import lax
from jax.experimental import pallas as pl
from jax.experimental.pallas import tpu as pltpu
from jax.experimental.pallas import tpu_sc as plsc

N = 10000
E = 320000
H = 128
C = 40
LANES = 16

NC = 2
NS = 16
NW = NC * NS
EPW = E // NW
CB = 128
NCHF = EPW // CB
TAIL = EPW - NCHF * CB
RPS = 624
RTAIL = N - NS * RPS
RB = 1000

_mesh = plsc.VectorSubcoreMesh(core_axis_name="c", subcore_axis_name="s")
_sc_params = pltpu.CompilerParams(needs_layout_passes=False)


@functools.partial(
    pl.kernel,
    out_type=jax.ShapeDtypeStruct((NC, N, H), jnp.float32),
    mesh=_mesh,
    compiler_params=_sc_params,
    scratch_types=[
        pltpu.VMEM((NCHF, CB), jnp.int32),
        pltpu.VMEM((1, TAIL), jnp.int32),
        pltpu.VMEM((NCHF, CB), jnp.int32),
        pltpu.VMEM((1, TAIL), jnp.int32),
        pltpu.VMEM((EPW,), jnp.float32),
        pltpu.VMEM((CB, H), jnp.float32),
        pltpu.VMEM((TAIL, H), jnp.float32),
        pltpu.VMEM_SHARED((N, H), jnp.float32),
        pltpu.SemaphoreType.DMA,
    ],
)
def _msg_kernel(x_hbm, srca_hbm, srcb_hbm, dsta_hbm, dstb_hbm, ce_hbm,
                zeros_hbm, part_hbm, srca_v, srcb_v, dsta_v, dstb_v, ce_v,
                rows_v, rowst_v, agg_sh, sem):
    c = lax.axis_index("c")
    s = lax.axis_index("s")
    wid = c * NS + s
    eb = wid * EPW
    rbase = s * RPS
    pltpu.sync_copy(srca_hbm.at[wid], srca_v)
    pltpu.sync_copy(srcb_hbm.at[wid], srcb_v)
    pltpu.sync_copy(dsta_hbm.at[wid], dsta_v)
    pltpu.sync_copy(dstb_hbm.at[wid], dstb_v)
    pltpu.sync_copy(ce_hbm.at[pl.ds(eb, EPW)], ce_v)
    pltpu.sync_copy(zeros_hbm.at[pl.ds(rbase, RPS)],
                    agg_sh.at[pl.ds(rbase, RPS)])

    @pl.when(s == NS - 1)
    def _():
        pltpu.sync_copy(zeros_hbm.at[pl.ds(NS * RPS, RTAIL)],
                        agg_sh.at[pl.ds(NS * RPS, RTAIL)])

    plsc.subcore_barrier()

    def _scale_rows(buf, nrows, woff):
        def row(r, carry):
            e16 = jnp.full((LANES,), woff + r, jnp.int32)
            w16 = plsc.load_gather(ce_v, [e16])
            for j in range(H // LANES):
                sl = pl.ds(j * LANES, LANES)
                buf[r, sl] = buf[r, sl] * w16
            return carry

        lax.fori_loop(0, nrows, row, 0)

    def chunk(ch, carry):
        pltpu.async_copy(x_hbm.at[srca_v.at[ch]], rows_v, sem).wait()
        _scale_rows(rows_v, CB, ch * CB)
        pltpu.sync_copy(rows_v, agg_sh.at[dsta_v.at[ch]], add=True)
        return carry

    lax.fori_loop(0, NCHF, chunk, 0)
    pltpu.async_copy(x_hbm.at[srcb_v.at[0]], rowst_v, sem).wait()
    _scale_rows(rowst_v, TAIL, NCHF * CB)
    pltpu.sync_copy(rowst_v, agg_sh.at[dstb_v.at[0]], add=True)
    plsc.subcore_barrier()
    pltpu.sync_copy(agg_sh.at[pl.ds(rbase, RPS)],
                    part_hbm.at[c, pl.ds(rbase, RPS)])

    @pl.when(s == NS - 1)
    def _():
        pltpu.sync_copy(agg_sh.at[pl.ds(NS * RPS, RTAIL)],
                        part_hbm.at[c, pl.ds(NS * RPS, RTAIL)])


def _tc_layer_body(part_ref, w_ref, b_ref, o_ref):
    t = part_ref[0] + part_ref[1]
    y = jnp.dot(t, w_ref[...], preferred_element_type=jnp.float32) + b_ref[...]
    o_ref[...] = jnp.where(y >= 0, y, 0.01 * y)


_tc_layer = pl.pallas_call(
    _tc_layer_body,
    grid=(N // RB,),
    in_specs=[
        pl.BlockSpec((2, RB, H), lambda i: (0, i, 0)),
        pl.BlockSpec((H, H), lambda i: (0, 0)),
        pl.BlockSpec((1, H), lambda i: (0, 0)),
    ],
    out_specs=pl.BlockSpec((RB, H), lambda i: (i, 0)),
    out_shape=jax.ShapeDtypeStruct((N, H), jnp.float32),
)


def _tc_last_body(part_ref, w6_ref, b6_ref, wl_ref, bl_ref, o_ref):
    t = part_ref[0] + part_ref[1]
    y = jnp.dot(t, w6_ref[...], preferred_element_type=jnp.float32) + b6_ref[...]
    h = jnp.where(y >= 0, y, 0.01 * y)
    o_ref[...] = jnp.dot(h, wl_ref[...],
                         preferred_element_type=jnp.float32) + bl_ref[...]


_tc_last = pl.pallas_call(
    _tc_last_body,
    grid=(N // RB,),
    in_specs=[
        pl.BlockSpec((2, RB, H), lambda i: (0, i, 0)),
        pl.BlockSpec((H, H), lambda i: (0, 0)),
        pl.BlockSpec((1, H), lambda i: (0, 0)),
        pl.BlockSpec((H, C), lambda i: (0, 0)),
        pl.BlockSpec((1, C), lambda i: (0, 0)),
    ],
    out_specs=pl.BlockSpec((RB, C), lambda i: (i, 0)),
    out_shape=jax.ShapeDtypeStruct((N, C), jnp.float32),
)


def kernel(in_feat, edge_index, edge_weight, W1, b1, W2, b2, W3, b3, W4, b4,
           W5, b5, W6, b6, Wl, bl):
    src = edge_index[0]
    dst = edge_index[1]
    src2 = src.reshape(NW, EPW)
    dst2 = dst.reshape(NW, EPW)
    srca = src2[:, :NCHF * CB].reshape(NW, NCHF, CB)
    srcb = src2[:, NCHF * CB:].reshape(NW, 1, TAIL)
    dsta = dst2[:, :NCHF * CB].reshape(NW, NCHF, CB)
    dstb = dst2[:, NCHF * CB:].reshape(NW, 1, TAIL)
    zeros_nh = jnp.zeros((N, H), jnp.float32)

    deg_out = jnp.clip(jnp.bincount(src, length=N).astype(jnp.float32), 1.0)
    deg_in = jnp.clip(jnp.bincount(dst, length=N).astype(jnp.float32), 1.0)
    do_inv = lax.rsqrt(deg_out)
    di_inv = lax.rsqrt(deg_in)
    ce = edge_weight * do_inv[src] * di_inv[dst]

    h = in_feat
    for (W, b) in ((W1, b1), (W2, b2), (W3, b3), (W4, b4), (W5, b5)):
        part = _msg_kernel(h, srca, srcb, dsta, dstb, ce, zeros_nh)
        h = _tc_layer(part, W, b[None, :])
    part = _msg_kernel(h, srca, srcb, dsta, dstb, ce, zeros_nh)
    return _tc_last(part, W6, b6[None, :], Wl, bl[None, :])
